# baseline (speedup 1.0000x reference)
import numpy as np
import jax
import jax.numpy as jnp
from jax.experimental import pallas as pl
from jax.experimental.pallas import tpu as pltpu

_MARGIN = 1.0
_B = 1024
_BLK = 128
_NB = _B // _BLK
_NPAIRS = _B * (_B - 1) // 2

_PAIRS = [(i, j) for i in range(_NB) for j in range(i, _NB)]
_NSTEPS = len(_PAIRS)
_II = np.array([p[0] for p in _PAIRS], dtype=np.int32)
_JJ = np.array([p[1] for p in _PAIRS], dtype=np.int32)


def _loss_kernel(ii_ref, jj_ref, ei_ref, ej_ref, ti_ref, tj_ref, out_ref):
    p = pl.program_id(0)
    bi = ii_ref[p]
    bj = jj_ref[p]
    ei = ei_ref[...]
    ej = ej_ref[...]
    gram = jnp.dot(ei, ej.T, preferred_element_type=jnp.float32)
    sqi = jnp.sum(ei * ei, axis=1, keepdims=True)
    sqj = jnp.sum(ej * ej, axis=1, keepdims=True)
    sqdist = jnp.maximum(sqi + sqj.T - 2.0 * gram, 1e-12)

    dist = sqdist * jax.lax.rsqrt(sqdist)
    neg = jnp.maximum(_MARGIN - dist, 0.0)
    neg = neg * neg

    eq = ti_ref[...].T == tj_ref[...]
    loss = jnp.where(eq, sqdist, neg)

    rows = jax.lax.broadcasted_iota(jnp.int32, (_BLK, _BLK), 0)
    cols = jax.lax.broadcasted_iota(jnp.int32, (_BLK, _BLK), 1)
    upper = cols - rows > _BLK * (bi - bj)
    partial = jnp.sum(jnp.where(upper, loss, 0.0))

    @pl.when(p == 0)
    def _init():
        out_ref[...] = jnp.zeros((1, 1), jnp.float32)

    out_ref[...] += (partial * (1.0 / _NPAIRS)).reshape(1, 1)


def kernel(embeddings, target):
    t = target.astype(jnp.int32).reshape(1, _B)
    grid_spec = pltpu.PrefetchScalarGridSpec(
        num_scalar_prefetch=2,
        grid=(_NSTEPS,),
        in_specs=[
            pl.BlockSpec((_BLK, 128), lambda p, ii, jj: (ii[p], 0)),
            pl.BlockSpec((_BLK, 128), lambda p, ii, jj: (jj[p], 0)),
            pl.BlockSpec((1, _BLK), lambda p, ii, jj: (0, ii[p])),
            pl.BlockSpec((1, _BLK), lambda p, ii, jj: (0, jj[p])),
        ],
        out_specs=pl.BlockSpec((1, 1), lambda p, ii, jj: (0, 0)),
    )
    out = pl.pallas_call(
        _loss_kernel,
        grid_spec=grid_spec,
        out_shape=jax.ShapeDtypeStruct((1, 1), jnp.float32),
    )(jnp.asarray(_II), jnp.asarray(_JJ), embeddings, embeddings, t, t)
    return out.reshape(())


# speedup vs baseline: 7.0074x; 7.0074x over previous
import jax
import jax.numpy as jnp
from jax.experimental import pallas as pl

_MARGIN = 1.0
_B = 1024
_BLK = 128
_NPAIRS = _B * (_B - 1) // 2


def _loss_kernel(e_ref, t_ref, out_ref):
    e = e_ref[...]
    t = t_ref[...]
    sq = jnp.sum(e * e, axis=1, keepdims=True)
    sqr = sq.T

    rows = jax.lax.broadcasted_iota(jnp.int32, (_BLK, _BLK), 0)
    cols = jax.lax.broadcasted_iota(jnp.int32, (_BLK, _BLK), 1)
    upper = cols > rows

    total = jnp.float32(0.0)
    for i in range(_B // _BLK):
        r0 = _BLK * i
        ei = e[r0:r0 + _BLK, :]
        ej = e[r0:, :]
        gram = jax.lax.dot_general(
            ei, ej, (((1,), (1,)), ((), ())),
            preferred_element_type=jnp.float32)
        sqd = jnp.maximum(sq[r0:r0 + _BLK, :] + sqr[:, r0:] - 2.0 * gram,
                          1e-12)
        dist = sqd * jax.lax.rsqrt(sqd)
        neg = jnp.maximum(_MARGIN - dist, 0.0)
        neg = neg * neg
        eq = t[:, r0:r0 + _BLK].T == t[:, r0:]
        loss = jnp.where(eq, sqd, neg)
        part = jnp.sum(jnp.where(upper, loss[:, :_BLK], 0.0))
        if r0 + _BLK < _B:
            part = part + jnp.sum(loss[:, _BLK:])
        total = total + part

    out_ref[...] = (total * (1.0 / _NPAIRS)).reshape(1, 1)


def kernel(embeddings, target):
    t = target.astype(jnp.int32).reshape(1, _B)
    out = pl.pallas_call(
        _loss_kernel,
        out_shape=jax.ShapeDtypeStruct((1, 1), jnp.float32),
    )(embeddings, t)
    return out.reshape(())


# speedup vs baseline: 7.0408x; 1.0048x over previous
import jax
import jax.numpy as jnp
from jax.experimental import pallas as pl

_MARGIN = 1.0
_B = 1024
_BLK = 128
_NPAIRS = _B * (_B - 1) // 2


def _loss_kernel(e_ref, t_ref, out_ref):
    e = e_ref[...]
    t = t_ref[...]
    sq = jnp.sum(e * e, axis=1, keepdims=True)
    sqr = sq.T
    em2 = e * -2.0

    rows = jax.lax.broadcasted_iota(jnp.int32, (_BLK, _BLK), 0)
    cols = jax.lax.broadcasted_iota(jnp.int32, (_BLK, _BLK), 1)
    upper = cols > rows

    total = jnp.float32(0.0)
    for i in range(_B // _BLK):
        r0 = _BLK * i
        ei = em2[r0:r0 + _BLK, :]
        ej = e[r0:, :]
        gram2 = jax.lax.dot_general(
            ei, ej, (((1,), (1,)), ((), ())),
            preferred_element_type=jnp.float32)
        sqd = jnp.maximum(sq[r0:r0 + _BLK, :] + sqr[:, r0:] + gram2,
                          1e-12)
        dist = sqd * jax.lax.rsqrt(sqd)
        neg = jnp.maximum(_MARGIN - dist, 0.0)
        neg = neg * neg
        eq = t[:, r0:r0 + _BLK].T == t[:, r0:]
        loss = jnp.where(eq, sqd, neg)
        part = jnp.sum(jnp.where(upper, loss[:, :_BLK], 0.0))
        if r0 + _BLK < _B:
            part = part + jnp.sum(loss[:, _BLK:])
        total = total + part

    out_ref[...] = (total * (1.0 / _NPAIRS)).reshape(1, 1)


def kernel(embeddings, target):
    t = target.astype(jnp.int32).reshape(1, _B)
    out = pl.pallas_call(
        _loss_kernel,
        out_shape=jax.ShapeDtypeStruct((1, 1), jnp.float32),
    )(embeddings, t)
    return out.reshape(())
